# Initial kernel scaffold; baseline (speedup 1.0000x reference)
import jax
import jax.numpy as jnp
from jax.experimental import pallas as pl


def kernel(inputs, token_table, pos_table):
    raise NotImplementedError("write your pallas kernel here")


# speedup vs baseline: 1.2555x; 1.2555x over previous
import functools

import jax
import jax.numpy as jnp
from jax import lax
from jax.experimental import pallas as pl
from jax.experimental.pallas import tpu as pltpu
from jax.experimental.pallas import tpu_sc as plsc

VOCAB = 1000000
SEQ_LEN = 200
EMBED_DIM = 32
BATCH = 4096

NC = 2
NS = 16
NW = NC * NS

TOTAL = BATCH * SEQ_LEN
PER_W = TOTAL // NW
ROWS_PER_CHUNK = 8 * SEQ_LEN
CHUNKS = PER_W // ROWS_PER_CHUNK
IDX_MINOR = 100
IDX_ROWS = ROWS_PER_CHUNK // IDX_MINOR


def _make_kernel():
    mesh = plsc.VectorSubcoreMesh(core_axis_name="c", subcore_axis_name="s")

    @functools.partial(
        pl.kernel,
        mesh=mesh,
        out_type=jax.ShapeDtypeStruct((TOTAL, EMBED_DIM), jnp.float32),
        compiler_params=pltpu.CompilerParams(use_tc_tiling_on_sc=False),
        scratch_types=[
            pltpu.VMEM((IDX_ROWS, IDX_MINOR), jnp.int32),
            pltpu.VMEM((ROWS_PER_CHUNK, EMBED_DIM), jnp.float32),
            pltpu.VMEM((SEQ_LEN, EMBED_DIM), jnp.float32),
            pltpu.SemaphoreType.DMA,
        ],
    )
    def k(idx_hbm, table_hbm, pos_hbm, out_hbm, idx_v, rows_v, pos_v, sem):
        wid = lax.axis_index("s") * NC + lax.axis_index("c")

        pltpu.sync_copy(pos_hbm, pos_v)

        def chunk_body(c, carry):
            base = pl.multiple_of(wid * PER_W + c * ROWS_PER_CHUNK, ROWS_PER_CHUNK)
            idx_row0 = pl.multiple_of(base // IDX_MINOR, IDX_ROWS)

            pltpu.sync_copy(idx_hbm.at[pl.ds(idx_row0, IDX_ROWS)], idx_v)

            for j in range(IDX_ROWS):
                pltpu.async_copy(
                    table_hbm.at[idx_v.at[j]],
                    rows_v.at[pl.ds(j * IDX_MINOR, IDX_MINOR)],
                    sem,
                ).wait()

            def pos_body(s, carry2):
                lo = pos_v[s, pl.ds(0, 16)]
                hi = pos_v[s, pl.ds(16, 16)]
                for b in range(8):
                    r = b * SEQ_LEN + s
                    plsc.addupdate(rows_v.at[r, pl.ds(0, 16)], lo)
                    plsc.addupdate(rows_v.at[r, pl.ds(16, 16)], hi)
                return carry2

            lax.fori_loop(0, SEQ_LEN, pos_body, 0)

            pltpu.sync_copy(rows_v, out_hbm.at[pl.ds(base, ROWS_PER_CHUNK)])
            return carry

        lax.fori_loop(0, CHUNKS, chunk_body, 0)

    return k


_sc_kernel = _make_kernel()


@jax.jit
def kernel(inputs, token_table, pos_table):
    idx = inputs.reshape(TOTAL // IDX_MINOR, IDX_MINOR).astype(jnp.int32)
    out = _sc_kernel(idx, token_table, pos_table)
    return out.reshape(BATCH, SEQ_LEN, EMBED_DIM)


# speedup vs baseline: 1.4851x; 1.1830x over previous
import functools

import jax
import jax.numpy as jnp
from jax import lax
from jax.experimental import pallas as pl
from jax.experimental.pallas import tpu as pltpu
from jax.experimental.pallas import tpu_sc as plsc

VOCAB = 1000000
SEQ_LEN = 200
EMBED_DIM = 32
BATCH = 4096

NC = 2
NS = 16
NW = NC * NS

TOTAL = BATCH * SEQ_LEN
PER_W = TOTAL // NW
SEQ_PER_CHUNK = 4
ROWS_PER_CHUNK = SEQ_PER_CHUNK * SEQ_LEN
CHUNKS = PER_W // ROWS_PER_CHUNK
IDX_MINOR = 100
IDX_ROWS_W = PER_W // IDX_MINOR
GATHERS = ROWS_PER_CHUNK // IDX_MINOR


def _make_kernel():
    mesh = plsc.VectorSubcoreMesh(core_axis_name="c", subcore_axis_name="s")

    @functools.partial(
        pl.kernel,
        mesh=mesh,
        out_type=jax.ShapeDtypeStruct((TOTAL, EMBED_DIM), jnp.float32),
        compiler_params=pltpu.CompilerParams(use_tc_tiling_on_sc=False),
        scratch_types=[
            pltpu.VMEM((IDX_ROWS_W, IDX_MINOR), jnp.int32),
            pltpu.VMEM((2, ROWS_PER_CHUNK, EMBED_DIM), jnp.float32),
            pltpu.VMEM((SEQ_LEN, EMBED_DIM), jnp.float32),
            pltpu.SemaphoreType.DMA((2,)),
        ],
    )
    def k(idx_hbm, table_hbm, pos_hbm, out_hbm, idx_v, rows_v, pos_v, gsem):
        wid = lax.axis_index("s") * NC + lax.axis_index("c")
        w_base = pl.multiple_of(wid * PER_W, PER_W)
        w_idx_row0 = pl.multiple_of(wid * IDX_ROWS_W, IDX_ROWS_W)

        pltpu.sync_copy(pos_hbm, pos_v)
        pltpu.sync_copy(idx_hbm.at[pl.ds(w_idx_row0, IDX_ROWS_W)], idx_v)

        def fire(c, p):
            for j in range(GATHERS):
                pltpu.async_copy(
                    table_hbm.at[idx_v.at[c * GATHERS + j]],
                    rows_v.at[p].at[pl.ds(j * IDX_MINOR, IDX_MINOR)],
                    gsem.at[p],
                )

        def drain(p):
            pltpu.make_async_copy(
                table_hbm.at[pl.ds(0, ROWS_PER_CHUNK)],
                rows_v.at[p],
                gsem.at[p],
            ).wait()

        def pos_add(p):
            def body(s, carry):
                lo = pos_v[s, pl.ds(0, 16)]
                hi = pos_v[s, pl.ds(16, 16)]
                for b in range(SEQ_PER_CHUNK):
                    r = b * SEQ_LEN + s
                    plsc.addupdate(rows_v.at[p].at[r, pl.ds(0, 16)], lo)
                    plsc.addupdate(rows_v.at[p].at[r, pl.ds(16, 16)], hi)
                return carry
            lax.fori_loop(0, SEQ_LEN, body, 0)

        def store(c, p):
            base = pl.multiple_of(w_base + c * ROWS_PER_CHUNK, ROWS_PER_CHUNK)
            pltpu.sync_copy(rows_v.at[p],
                            out_hbm.at[pl.ds(base, ROWS_PER_CHUNK)])

        fire(0, 0)
        for c in range(CHUNKS):
            p = c % 2
            if c + 1 < CHUNKS:
                fire(c + 1, 1 - p)
            drain(p)
            pos_add(p)
            store(c, p)

    return k


_sc_kernel = _make_kernel()


@jax.jit
def kernel(inputs, token_table, pos_table):
    idx = inputs.reshape(TOTAL // IDX_MINOR, IDX_MINOR).astype(jnp.int32)
    out = _sc_kernel(idx, token_table, pos_table)
    return out.reshape(BATCH, SEQ_LEN, EMBED_DIM)


# speedup vs baseline: 2.1857x; 1.4717x over previous
import functools

import jax
import jax.numpy as jnp
from jax import lax
from jax.experimental import pallas as pl
from jax.experimental.pallas import tpu as pltpu
from jax.experimental.pallas import tpu_sc as plsc

VOCAB = 1000000
SEQ_LEN = 200
EMBED_DIM = 32
BATCH = 4096

NC = 2
NS = 16
NW = NC * NS
L = 16

BTILE = 128
NJ = BATCH // BTILE
UNITS = SEQ_LEN * NJ
UNITS_PER_W = UNITS // NW
IDX_PER_W = UNITS_PER_W * BTILE
TPITCH = BTILE + 1


def _make_kernel():
    mesh = plsc.VectorSubcoreMesh(core_axis_name="c", subcore_axis_name="s")

    @functools.partial(
        pl.kernel,
        mesh=mesh,
        out_type=jax.ShapeDtypeStruct((SEQ_LEN * 4, NJ, 8, BTILE),
                                      jnp.float32),
        compiler_params=pltpu.CompilerParams(use_tc_tiling_on_sc=False,
                                             needs_layout_passes=False),
        scratch_types=[
            pltpu.VMEM((IDX_PER_W,), jnp.int32),
            pltpu.VMEM((2, BTILE, EMBED_DIM), jnp.float32),
            pltpu.VMEM((2, EMBED_DIM, TPITCH), jnp.float32),
            pltpu.VMEM((SEQ_LEN, EMBED_DIM), jnp.float32),
            pltpu.SemaphoreType.DMA((2,)),
            pltpu.SemaphoreType.DMA((2,)),
        ],
    )
    def k(idx_hbm, table_hbm, pos_hbm, out_hbm, idx_v, rows_v, trans_v,
          pos_v, gsem, osem):
        wid = lax.axis_index("s") * NC + lax.axis_index("c")
        g0 = wid * UNITS_PER_W

        pltpu.sync_copy(
            idx_hbm.at[pl.ds(pl.multiple_of(wid * IDX_PER_W, IDX_PER_W),
                             IDX_PER_W)],
            idx_v)
        pltpu.sync_copy(pos_hbm, pos_v)

        lane = lax.broadcasted_iota(jnp.int32, (L,), 0)

        def fire(uu, p):
            pltpu.async_copy(
                table_hbm.at[idx_v.at[pl.ds(uu * BTILE, BTILE)]],
                rows_v.at[p],
                gsem.at[p],
            )

        def drain_gather(p):
            pltpu.make_async_copy(
                table_hbm.at[pl.ds(0, BTILE)], rows_v.at[p], gsem.at[p]
            ).wait()

        def transform(uu, p):
            s = (g0 + uu) // NJ
            plo = pos_v[s, pl.ds(0, L)]
            phi = pos_v[s, pl.ds(L, L)]

            def rbody(rr, carry):
                for q in range(4):
                    r = rr * 4 + q
                    rcol = jnp.full((L,), 0, jnp.int32) + r
                    lo = rows_v[p, r, pl.ds(0, L)] + plo
                    hi = rows_v[p, r, pl.ds(L, L)] + phi
                    plsc.store_scatter(trans_v.at[p], [lane, rcol], lo)
                    plsc.store_scatter(trans_v.at[p], [lane + L, rcol], hi)
                return carry

            lax.fori_loop(0, BTILE // 4, rbody, 0)

        def fire_stores(uu, p):
            g = g0 + uu
            s = g // NJ
            j = g % NJ
            for i in range(4):
                pltpu.async_copy(
                    trans_v.at[p].at[pl.ds(i * 8, 8), pl.ds(0, BTILE)],
                    out_hbm.at[s * 4 + i, j],
                    osem.at[p],
                )

        def drain_stores(p):
            for i in range(4):
                pltpu.make_async_copy(
                    trans_v.at[p].at[pl.ds(i * 8, 8), pl.ds(0, BTILE)],
                    out_hbm.at[0, 0],
                    osem.at[p],
                ).wait()

        fire(0, 0)

        def pair(h, carry):
            uu = h * 2
            @pl.when(uu + 1 < UNITS_PER_W)
            def _():
                fire(uu + 1, 1)
            drain_gather(0)
            @pl.when(uu >= 2)
            def _():
                drain_stores(0)
            transform(uu, 0)
            fire_stores(uu, 0)
            @pl.when(uu + 2 < UNITS_PER_W)
            def _():
                fire(uu + 2, 0)
            drain_gather(1)
            @pl.when(uu >= 2)
            def _():
                drain_stores(1)
            transform(uu + 1, 1)
            fire_stores(uu + 1, 1)
            return carry

        lax.fori_loop(0, UNITS_PER_W // 2, pair, 0)
        drain_stores(0)
        drain_stores(1)

    return k


_sc_kernel = _make_kernel()


@jax.jit
def kernel(inputs, token_table, pos_table):
    idx = inputs.astype(jnp.int32).T.reshape(-1)
    tbl = lax.optimization_barrier(token_table.reshape(-1))
    tbl = tbl.reshape(VOCAB, EMBED_DIM)
    out4 = _sc_kernel(idx, tbl, pos_table)
    o5 = out4.reshape(SEQ_LEN, 4, NJ, 8, BTILE)
    return o5.transpose(2, 4, 0, 1, 3).reshape(BATCH, SEQ_LEN, EMBED_DIM)
